# baseline (speedup 1.0000x reference)
import functools

import jax
import jax.numpy as jnp
from jax import lax
from jax.experimental import pallas as pl
from jax.experimental.pallas import tpu as pltpu
from jax.experimental.pallas import tpu_sc as plsc

N = 10000
E = 320000
D = 128
NC, NS = 2, 16
NW = NC * NS
CHUNK = 128
ITERS = 80
EPT = CHUNK * ITERS
E_PAD = EPT * NW
N_PAD = 10240
ROWS_PT = N_PAD // NS
NBUF = 2

_MESH = dict(core_axis_name="c", subcore_axis_name="s", num_cores=NC,
             num_subcores=NS)


def _deg_body(src_hbm, dst_hbm, dout_hbm, din_hbm,
              srcv, dstv, dout_acc, din_acc, sem):
  c = lax.axis_index("c")
  s = lax.axis_index("s")
  tile = c * NS + s

  def z(i, carry):
    dout_acc[pl.ds(i * 16, 16)] = jnp.zeros((16,), jnp.float32)
    din_acc[pl.ds(i * 16, 16)] = jnp.zeros((16,), jnp.float32)
    return carry

  lax.fori_loop(0, N_PAD // 16, z, 0)

  base = pl.multiple_of(tile * EPT, 8)
  pltpu.sync_copy(src_hbm.at[pl.ds(base, EPT)], srcv)
  pltpu.sync_copy(dst_hbm.at[pl.ds(base, EPT)], dstv)
  ones = jnp.ones((16,), jnp.float32)

  def it(i, carry):
    plsc.addupdate_scatter(dout_acc, [srcv[pl.ds(i * 16, 16)]], ones)
    plsc.addupdate_scatter(din_acc, [dstv[pl.ds(i * 16, 16)]], ones)
    return carry

  lax.fori_loop(0, EPT // 16, it, 0)
  pltpu.sync_copy(dout_acc, dout_hbm.at[tile])
  pltpu.sync_copy(din_acc, din_hbm.at[tile])


@functools.cache
def _deg_kernel():
  return pl.kernel(
      _deg_body,
      out_type=(jax.ShapeDtypeStruct((NW, N_PAD), jnp.float32),
                jax.ShapeDtypeStruct((NW, N_PAD), jnp.float32)),
      mesh=plsc.VectorSubcoreMesh(**_MESH),
      scratch_types=[
          pltpu.VMEM((EPT,), jnp.int32),
          pltpu.VMEM((EPT,), jnp.int32),
          pltpu.VMEM((N_PAD,), jnp.float32),
          pltpu.VMEM((N_PAD,), jnp.float32),
          pltpu.SemaphoreType.DMA,
      ],
      compiler_params=pltpu.CompilerParams(needs_layout_passes=False),
  )


def _agg_body(h_hbm, src_hbm, dst_hbm, zeros_hbm, out_hbm,
              dstb, rows, acc, sems_g):
  c = lax.axis_index("c")
  s = lax.axis_index("s")
  tile = c * NS + s
  r0 = pl.multiple_of(s * ROWS_PT, 8)
  pltpu.sync_copy(zeros_hbm.at[pl.ds(r0, ROWS_PT)], acc.at[pl.ds(r0, ROWS_PT)])
  e0 = pl.multiple_of(tile * EPT, 8)
  plsc.subcore_barrier()

  def it(i, carry):
    base = pl.multiple_of(e0 + i * CHUNK, 8)
    pltpu.sync_copy(src_hbm.at[pl.ds(base, CHUNK)], dstb[0])
    pltpu.async_copy(h_hbm.at[dstb[0]], rows[0], sems_g[0]).wait()
    pltpu.sync_copy(dst_hbm.at[pl.ds(base, CHUNK)], dstb[1])
    pltpu.sync_copy(rows[0], acc.at[dstb[1]], add=True)
    return carry

  lax.fori_loop(0, ITERS, it, 0)

  plsc.subcore_barrier()
  pltpu.sync_copy(acc.at[pl.ds(r0, ROWS_PT)], out_hbm.at[c, pl.ds(r0, ROWS_PT)])


@functools.cache
def _agg_kernel():
  return pl.kernel(
      _agg_body,
      out_type=jax.ShapeDtypeStruct((NC, N_PAD, D), jnp.float32),
      mesh=plsc.VectorSubcoreMesh(**_MESH),
      scratch_types=[
          [pltpu.VMEM((CHUNK,), jnp.int32)] * 2,
          [pltpu.VMEM((CHUNK, D), jnp.float32)] * 1,
          pltpu.VMEM_SHARED((N_PAD, D), jnp.float32),
          [pltpu.SemaphoreType.DMA] * 1,
      ],
  )


def _scale_from_parts(parts_ref):
  d = jnp.sum(parts_ref[...], axis=0)[:, None]
  return lax.rsqrt(jnp.maximum(d, 1.0))


def _tc_pre_body(feat_ref, dop_ref, h_ref):
  dos = _scale_from_parts(dop_ref)
  h_ref[:N] = feat_ref[...] * dos[:N]
  h_ref[N:] = jnp.zeros((N_PAD - N, D), jnp.float32)


def _tc_mid_body(p_ref, w_ref, b_ref, dop_ref, dip_ref, h_ref):
  agg = p_ref[0] + p_ref[1]
  dis = _scale_from_parts(dip_ref)
  dos = _scale_from_parts(dop_ref)
  rst = jnp.dot(agg, w_ref[...], preferred_element_type=jnp.float32)
  x = jnp.maximum(rst * dis + b_ref[...], 0.0)
  h_ref[:N] = x[:N] * dos[:N]
  h_ref[N:] = jnp.zeros((N_PAD - N, D), jnp.float32)


def _tc_post_body(p_ref, w_ref, b_ref, dip_ref,
                  wm0_ref, bm0_ref, wm1_ref, bm1_ref,
                  wm2_ref, bm2_ref, wm3_ref, bm3_ref, y_ref):
  agg = p_ref[0, :N, :] + p_ref[1, :N, :]
  dis = _scale_from_parts(dip_ref)[:N]
  rst = jnp.dot(agg, w_ref[...], preferred_element_type=jnp.float32)
  x = jnp.maximum(rst * dis + b_ref[...], 0.0)
  hg = jnp.mean(x, axis=0, keepdims=True)
  y = jnp.maximum(
      jnp.dot(hg, wm0_ref[...], preferred_element_type=jnp.float32)
      + bm0_ref[...], 0.0)
  y = jnp.maximum(
      jnp.dot(y, wm1_ref[...], preferred_element_type=jnp.float32)
      + bm1_ref[...], 0.0)
  y = jnp.maximum(
      jnp.dot(y, wm2_ref[...], preferred_element_type=jnp.float32)
      + bm2_ref[...], 0.0)
  y_ref[...] = (jnp.sum(y * wm3_ref[...].reshape(1, 16), axis=1, keepdims=True)
                + bm3_ref[...])


def kernel(feat, edge_index, W1, b1, W2, b2,
           Wm0, bm0, Wm1, bm1, Wm2, bm2, Wm3, bm3):
  pad = N + jnp.arange(E_PAD - E, dtype=jnp.int32) % (N_PAD - N)
  src = jnp.concatenate([edge_index[0], pad])
  dst = jnp.concatenate([edge_index[1], pad])
  zeros_agg = jnp.zeros((N_PAD, D), jnp.float32)

  dout_parts, din_parts = _deg_kernel()(src, dst)

  h1 = pl.pallas_call(
      _tc_pre_body,
      out_shape=jax.ShapeDtypeStruct((N_PAD, D), jnp.float32),
  )(feat, dout_parts)

  agg1 = _agg_kernel()(h1, src, dst, zeros_agg)

  h2 = pl.pallas_call(
      _tc_mid_body,
      out_shape=jax.ShapeDtypeStruct((N_PAD, D), jnp.float32),
  )(agg1, W1, b1.reshape(1, D), dout_parts, din_parts)

  agg2 = _agg_kernel()(h2, src, dst, zeros_agg)

  y = pl.pallas_call(
      _tc_post_body,
      out_shape=jax.ShapeDtypeStruct((1, 1), jnp.float32),
  )(agg2, W2, b2.reshape(1, D), din_parts,
    Wm0, bm0.reshape(1, -1), Wm1, bm1.reshape(1, -1),
    Wm2, bm2.reshape(1, -1), Wm3, bm3.reshape(1, -1))
  return y


# speedup vs baseline: 1.9851x; 1.9851x over previous
import functools

import jax
import jax.numpy as jnp
from jax import lax
from jax.experimental import pallas as pl
from jax.experimental.pallas import tpu as pltpu
from jax.experimental.pallas import tpu_sc as plsc

N = 10000
E = 320000
D = 128
NC, NS = 2, 16
NW = NC * NS
CHUNK = 128
ITERS = 80
EPT = CHUNK * ITERS
E_PAD = EPT * NW
N_PAD = 10240
ROWS_PT = N_PAD // NS
NBUF = 2

_MESH = dict(core_axis_name="c", subcore_axis_name="s", num_cores=NC,
             num_subcores=NS)


def _deg_body(src_hbm, dst_hbm, dout_hbm, din_hbm,
              srcv, dstv, dout_acc, din_acc, sem):
  c = lax.axis_index("c")
  s = lax.axis_index("s")
  tile = c * NS + s

  def z(i, carry):
    dout_acc[pl.ds(i * 16, 16)] = jnp.zeros((16,), jnp.float32)
    din_acc[pl.ds(i * 16, 16)] = jnp.zeros((16,), jnp.float32)
    return carry

  lax.fori_loop(0, N_PAD // 16, z, 0)

  base = pl.multiple_of(tile * EPT, 8)
  pltpu.sync_copy(src_hbm.at[pl.ds(base, EPT)], srcv)
  pltpu.sync_copy(dst_hbm.at[pl.ds(base, EPT)], dstv)
  ones = jnp.ones((16,), jnp.float32)

  def it(i, carry):
    plsc.addupdate_scatter(dout_acc, [srcv[pl.ds(i * 16, 16)]], ones)
    plsc.addupdate_scatter(din_acc, [dstv[pl.ds(i * 16, 16)]], ones)
    return carry

  lax.fori_loop(0, EPT // 16, it, 0)
  pltpu.sync_copy(dout_acc, dout_hbm.at[tile])
  pltpu.sync_copy(din_acc, din_hbm.at[tile])


@functools.cache
def _deg_kernel():
  return pl.kernel(
      _deg_body,
      out_type=(jax.ShapeDtypeStruct((NW, N_PAD), jnp.float32),
                jax.ShapeDtypeStruct((NW, N_PAD), jnp.float32)),
      mesh=plsc.VectorSubcoreMesh(**_MESH),
      scratch_types=[
          pltpu.VMEM((EPT,), jnp.int32),
          pltpu.VMEM((EPT,), jnp.int32),
          pltpu.VMEM((N_PAD,), jnp.float32),
          pltpu.VMEM((N_PAD,), jnp.float32),
          pltpu.SemaphoreType.DMA,
      ],
      compiler_params=pltpu.CompilerParams(needs_layout_passes=False),
  )


def _agg_body(h_hbm, src_hbm, dst_hbm, zeros_hbm, out_hbm,
              srcv, dstb, rows, acc, sems_g, sems_i):
  c = lax.axis_index("c")
  s = lax.axis_index("s")
  tile = c * NS + s
  r0 = pl.multiple_of(s * ROWS_PT, 8)
  pltpu.sync_copy(zeros_hbm.at[pl.ds(r0, ROWS_PT)], acc.at[pl.ds(r0, ROWS_PT)])
  e0 = pl.multiple_of(tile * EPT, 8)
  pltpu.sync_copy(src_hbm.at[pl.ds(e0, EPT)], srcv)
  plsc.subcore_barrier()

  def gather(i, b):
    idx = pl.multiple_of(i * CHUNK, 8)
    pltpu.async_copy(h_hbm.at[srcv.at[pl.ds(idx, CHUNK)]], rows[b], sems_g[b])

  def wait_gather(i, b):
    idx = pl.multiple_of(i * CHUNK, 8)
    pltpu.make_async_copy(h_hbm.at[srcv.at[pl.ds(idx, CHUNK)]], rows[b],
                          sems_g[b]).wait()

  def dst_fetch(i, b):
    base = pl.multiple_of(e0 + i * CHUNK, 8)
    pltpu.async_copy(dst_hbm.at[pl.ds(base, CHUNK)], dstb[b], sems_i[b])

  def wait_dst(i, b):
    base = pl.multiple_of(e0 + i * CHUNK, 8)
    pltpu.make_async_copy(dst_hbm.at[pl.ds(base, CHUNK)], dstb[b],
                          sems_i[b]).wait()

  for b in range(NBUF):
    dst_fetch(b, b)
    gather(b, b)

  def it(outer, carry):
    for b in range(NBUF):
      i = outer * NBUF + b
      wait_gather(i, b)
      wait_dst(i, b)
      pltpu.sync_copy(rows[b], acc.at[dstb[b]], add=True)
      dst_fetch(i + NBUF, b)
      gather(i + NBUF, b)
    return carry

  lax.fori_loop(0, (ITERS - NBUF) // NBUF, it, 0)
  for b in range(NBUF):
    i = ITERS - NBUF + b
    wait_gather(i, b)
    wait_dst(i, b)
    pltpu.sync_copy(rows[b], acc.at[dstb[b]], add=True)

  plsc.subcore_barrier()
  pltpu.sync_copy(acc.at[pl.ds(r0, ROWS_PT)], out_hbm.at[c, pl.ds(r0, ROWS_PT)])


@functools.cache
def _agg_kernel():
  return pl.kernel(
      _agg_body,
      out_type=jax.ShapeDtypeStruct((NC, N_PAD, D), jnp.float32),
      mesh=plsc.VectorSubcoreMesh(**_MESH),
      scratch_types=[
          pltpu.VMEM((EPT,), jnp.int32),
          [pltpu.VMEM((CHUNK,), jnp.int32)] * NBUF,
          [pltpu.VMEM((CHUNK, D), jnp.float32)] * NBUF,
          pltpu.VMEM_SHARED((N_PAD, D), jnp.float32),
          [pltpu.SemaphoreType.DMA] * NBUF,
          [pltpu.SemaphoreType.DMA] * NBUF,
      ],
  )


def _scale_from_parts(parts_ref):
  d = jnp.sum(parts_ref[...], axis=0)[:, None]
  return lax.rsqrt(jnp.maximum(d, 1.0))


def _tc_pre_body(feat_ref, dop_ref, h_ref):
  dos = _scale_from_parts(dop_ref)
  h_ref[:N] = feat_ref[...] * dos[:N]
  h_ref[N:] = jnp.zeros((N_PAD - N, D), jnp.float32)


def _tc_mid_body(p_ref, w_ref, b_ref, dop_ref, dip_ref, h_ref):
  agg = p_ref[0] + p_ref[1]
  dis = _scale_from_parts(dip_ref)
  dos = _scale_from_parts(dop_ref)
  rst = jnp.dot(agg, w_ref[...], preferred_element_type=jnp.float32)
  x = jnp.maximum(rst * dis + b_ref[...], 0.0)
  h_ref[:N] = x[:N] * dos[:N]
  h_ref[N:] = jnp.zeros((N_PAD - N, D), jnp.float32)


def _tc_post_body(p_ref, w_ref, b_ref, dip_ref,
                  wm0_ref, bm0_ref, wm1_ref, bm1_ref,
                  wm2_ref, bm2_ref, wm3_ref, bm3_ref, y_ref):
  agg = p_ref[0, :N, :] + p_ref[1, :N, :]
  dis = _scale_from_parts(dip_ref)[:N]
  rst = jnp.dot(agg, w_ref[...], preferred_element_type=jnp.float32)
  x = jnp.maximum(rst * dis + b_ref[...], 0.0)
  hg = jnp.mean(x, axis=0, keepdims=True)
  y = jnp.maximum(
      jnp.dot(hg, wm0_ref[...], preferred_element_type=jnp.float32)
      + bm0_ref[...], 0.0)
  y = jnp.maximum(
      jnp.dot(y, wm1_ref[...], preferred_element_type=jnp.float32)
      + bm1_ref[...], 0.0)
  y = jnp.maximum(
      jnp.dot(y, wm2_ref[...], preferred_element_type=jnp.float32)
      + bm2_ref[...], 0.0)
  y_ref[...] = (jnp.sum(y * wm3_ref[...].reshape(1, 16), axis=1, keepdims=True)
                + bm3_ref[...])


def kernel(feat, edge_index, W1, b1, W2, b2,
           Wm0, bm0, Wm1, bm1, Wm2, bm2, Wm3, bm3):
  pad = N + jnp.arange(E_PAD - E, dtype=jnp.int32) % (N_PAD - N)
  src = jnp.concatenate([edge_index[0], pad])
  dst = jnp.concatenate([edge_index[1], pad])
  zeros_agg = jnp.zeros((N_PAD, D), jnp.float32)

  dout_parts, din_parts = _deg_kernel()(src, dst)

  h1 = pl.pallas_call(
      _tc_pre_body,
      out_shape=jax.ShapeDtypeStruct((N_PAD, D), jnp.float32),
  )(feat, dout_parts)

  agg1 = _agg_kernel()(h1, src, dst, zeros_agg)

  h2 = pl.pallas_call(
      _tc_mid_body,
      out_shape=jax.ShapeDtypeStruct((N_PAD, D), jnp.float32),
  )(agg1, W1, b1.reshape(1, D), dout_parts, din_parts)

  agg2 = _agg_kernel()(h2, src, dst, zeros_agg)

  y = pl.pallas_call(
      _tc_post_body,
      out_shape=jax.ShapeDtypeStruct((1, 1), jnp.float32),
  )(agg2, W2, b2.reshape(1, D), din_parts,
    Wm0, bm0.reshape(1, -1), Wm1, bm1.reshape(1, -1),
    Wm2, bm2.reshape(1, -1), Wm3, bm3.reshape(1, -1))
  return y


# speedup vs baseline: 2.1952x; 1.1058x over previous
import functools

import jax
import jax.numpy as jnp
from jax import lax
from jax.experimental import pallas as pl
from jax.experimental.pallas import tpu as pltpu
from jax.experimental.pallas import tpu_sc as plsc

N = 10000
E = 320000
D = 128
NC, NS = 2, 16
NW = NC * NS
CHUNK = 96
ITERS = 105
EPT = CHUNK * ITERS
E_PAD = EPT * NW
N_PAD = 10112
ROWS_PT = N_PAD // NS
NBUF = 3

_MESH = dict(core_axis_name="c", subcore_axis_name="s", num_cores=NC,
             num_subcores=NS)


def _deg_body(src_hbm, dst_hbm, dout_hbm, din_hbm,
              srcv, dstv, dout_acc, din_acc, sem):
  c = lax.axis_index("c")
  s = lax.axis_index("s")
  tile = c * NS + s

  def z(i, carry):
    dout_acc[pl.ds(i * 16, 16)] = jnp.zeros((16,), jnp.float32)
    din_acc[pl.ds(i * 16, 16)] = jnp.zeros((16,), jnp.float32)
    return carry

  lax.fori_loop(0, N_PAD // 16, z, 0)

  base = pl.multiple_of(tile * EPT, 8)
  pltpu.sync_copy(src_hbm.at[pl.ds(base, EPT)], srcv)
  pltpu.sync_copy(dst_hbm.at[pl.ds(base, EPT)], dstv)
  ones = jnp.ones((16,), jnp.float32)

  def it(i, carry):
    plsc.addupdate_scatter(dout_acc, [srcv[pl.ds(i * 16, 16)]], ones)
    plsc.addupdate_scatter(din_acc, [dstv[pl.ds(i * 16, 16)]], ones)
    return carry

  lax.fori_loop(0, EPT // 16, it, 0)
  pltpu.sync_copy(dout_acc, dout_hbm.at[tile])
  pltpu.sync_copy(din_acc, din_hbm.at[tile])


@functools.cache
def _deg_kernel():
  return pl.kernel(
      _deg_body,
      out_type=(jax.ShapeDtypeStruct((NW, N_PAD), jnp.float32),
                jax.ShapeDtypeStruct((NW, N_PAD), jnp.float32)),
      mesh=plsc.VectorSubcoreMesh(**_MESH),
      scratch_types=[
          pltpu.VMEM((EPT,), jnp.int32),
          pltpu.VMEM((EPT,), jnp.int32),
          pltpu.VMEM((N_PAD,), jnp.float32),
          pltpu.VMEM((N_PAD,), jnp.float32),
          pltpu.SemaphoreType.DMA,
      ],
      compiler_params=pltpu.CompilerParams(needs_layout_passes=False),
  )


def _agg_body(h_hbm, src_hbm, dst_hbm, zeros_hbm, out_hbm,
              srcv, dstb, rows, acc, sems_g, sems_i):
  c = lax.axis_index("c")
  s = lax.axis_index("s")
  tile = c * NS + s
  r0 = pl.multiple_of(s * ROWS_PT, 8)
  pltpu.sync_copy(zeros_hbm.at[pl.ds(r0, ROWS_PT)], acc.at[pl.ds(r0, ROWS_PT)])
  e0 = pl.multiple_of(tile * EPT, 8)
  pltpu.sync_copy(src_hbm.at[pl.ds(e0, EPT)], srcv)
  plsc.subcore_barrier()

  def gather(i, b):
    idx = pl.multiple_of(i * CHUNK, 8)
    pltpu.async_copy(h_hbm.at[srcv.at[pl.ds(idx, CHUNK)]], rows[b], sems_g[b])

  def wait_gather(i, b):
    idx = pl.multiple_of(i * CHUNK, 8)
    pltpu.make_async_copy(h_hbm.at[srcv.at[pl.ds(idx, CHUNK)]], rows[b],
                          sems_g[b]).wait()

  def dst_fetch(i, b):
    base = pl.multiple_of(e0 + i * CHUNK, 8)
    pltpu.async_copy(dst_hbm.at[pl.ds(base, CHUNK)], dstb[b], sems_i[b])

  def wait_dst(i, b):
    base = pl.multiple_of(e0 + i * CHUNK, 8)
    pltpu.make_async_copy(dst_hbm.at[pl.ds(base, CHUNK)], dstb[b],
                          sems_i[b]).wait()

  for b in range(NBUF):
    dst_fetch(b, b)
    gather(b, b)

  def it(outer, carry):
    for b in range(NBUF):
      i = outer * NBUF + b
      wait_gather(i, b)
      wait_dst(i, b)
      pltpu.sync_copy(rows[b], acc.at[dstb[b]], add=True)
      dst_fetch(i + NBUF, b)
      gather(i + NBUF, b)
    return carry

  lax.fori_loop(0, (ITERS - NBUF) // NBUF, it, 0)
  for b in range(NBUF):
    i = ITERS - NBUF + b
    wait_gather(i, b)
    wait_dst(i, b)
    pltpu.sync_copy(rows[b], acc.at[dstb[b]], add=True)

  plsc.subcore_barrier()
  pltpu.sync_copy(acc.at[pl.ds(r0, ROWS_PT)], out_hbm.at[c, pl.ds(r0, ROWS_PT)])


@functools.cache
def _agg_kernel():
  return pl.kernel(
      _agg_body,
      out_type=jax.ShapeDtypeStruct((NC, N_PAD, D), jnp.float32),
      mesh=plsc.VectorSubcoreMesh(**_MESH),
      scratch_types=[
          pltpu.VMEM((EPT,), jnp.int32),
          [pltpu.VMEM((CHUNK,), jnp.int32)] * NBUF,
          [pltpu.VMEM((CHUNK, D), jnp.float32)] * NBUF,
          pltpu.VMEM_SHARED((N_PAD, D), jnp.float32),
          [pltpu.SemaphoreType.DMA] * NBUF,
          [pltpu.SemaphoreType.DMA] * NBUF,
      ],
  )


def _scale_from_parts(parts_ref):
  d = jnp.sum(parts_ref[...], axis=0)[:, None]
  return lax.rsqrt(jnp.maximum(d, 1.0))


def _tc_pre_body(feat_ref, dop_ref, h_ref):
  dos = _scale_from_parts(dop_ref)
  h_ref[:N] = feat_ref[...] * dos[:N]
  h_ref[N:] = jnp.zeros((N_PAD - N, D), jnp.float32)


def _tc_mid_body(p_ref, w_ref, b_ref, dop_ref, dip_ref, h_ref):
  agg = p_ref[0] + p_ref[1]
  dis = _scale_from_parts(dip_ref)
  dos = _scale_from_parts(dop_ref)
  rst = jnp.dot(agg, w_ref[...], preferred_element_type=jnp.float32)
  x = jnp.maximum(rst * dis + b_ref[...], 0.0)
  h_ref[:N] = x[:N] * dos[:N]
  h_ref[N:] = jnp.zeros((N_PAD - N, D), jnp.float32)


def _tc_post_body(p_ref, w_ref, b_ref, dip_ref,
                  wm0_ref, bm0_ref, wm1_ref, bm1_ref,
                  wm2_ref, bm2_ref, wm3_ref, bm3_ref, y_ref):
  agg = p_ref[0, :N, :] + p_ref[1, :N, :]
  dis = _scale_from_parts(dip_ref)[:N]
  rst = jnp.dot(agg, w_ref[...], preferred_element_type=jnp.float32)
  x = jnp.maximum(rst * dis + b_ref[...], 0.0)
  hg = jnp.mean(x, axis=0, keepdims=True)
  y = jnp.maximum(
      jnp.dot(hg, wm0_ref[...], preferred_element_type=jnp.float32)
      + bm0_ref[...], 0.0)
  y = jnp.maximum(
      jnp.dot(y, wm1_ref[...], preferred_element_type=jnp.float32)
      + bm1_ref[...], 0.0)
  y = jnp.maximum(
      jnp.dot(y, wm2_ref[...], preferred_element_type=jnp.float32)
      + bm2_ref[...], 0.0)
  y_ref[...] = (jnp.sum(y * wm3_ref[...].reshape(1, 16), axis=1, keepdims=True)
                + bm3_ref[...])


def kernel(feat, edge_index, W1, b1, W2, b2,
           Wm0, bm0, Wm1, bm1, Wm2, bm2, Wm3, bm3):
  pad = N + jnp.arange(E_PAD - E, dtype=jnp.int32) % (N_PAD - N)
  src = jnp.concatenate([edge_index[0], pad])
  dst = jnp.concatenate([edge_index[1], pad])
  zeros_agg = jnp.zeros((N_PAD, D), jnp.float32)

  dout_parts, din_parts = _deg_kernel()(src, dst)

  h1 = pl.pallas_call(
      _tc_pre_body,
      out_shape=jax.ShapeDtypeStruct((N_PAD, D), jnp.float32),
  )(feat, dout_parts)

  agg1 = _agg_kernel()(h1, src, dst, zeros_agg)

  h2 = pl.pallas_call(
      _tc_mid_body,
      out_shape=jax.ShapeDtypeStruct((N_PAD, D), jnp.float32),
  )(agg1, W1, b1.reshape(1, D), dout_parts, din_parts)

  agg2 = _agg_kernel()(h2, src, dst, zeros_agg)

  y = pl.pallas_call(
      _tc_post_body,
      out_shape=jax.ShapeDtypeStruct((1, 1), jnp.float32),
  )(agg2, W2, b2.reshape(1, D), din_parts,
    Wm0, bm0.reshape(1, -1), Wm1, bm1.reshape(1, -1),
    Wm2, bm2.reshape(1, -1), Wm3, bm3.reshape(1, -1))
  return y


# speedup vs baseline: 2.2329x; 1.0172x over previous
import functools

import jax
import jax.numpy as jnp
from jax import lax
from jax.experimental import pallas as pl
from jax.experimental.pallas import tpu as pltpu
from jax.experimental.pallas import tpu_sc as plsc

N = 10000
E = 320000
D = 128
NC, NS = 2, 16
NW = NC * NS
CHUNK = 72
ITERS = 140
EPT = CHUNK * ITERS
E_PAD = EPT * NW
N_PAD = 10112
ROWS_PT = N_PAD // NS
NBUF = 4

_MESH = dict(core_axis_name="c", subcore_axis_name="s", num_cores=NC,
             num_subcores=NS)


def _deg_body(src_hbm, dst_hbm, dout_hbm, din_hbm,
              srcv, dstv, dout_acc, din_acc, sem):
  c = lax.axis_index("c")
  s = lax.axis_index("s")
  tile = c * NS + s

  def z(i, carry):
    dout_acc[pl.ds(i * 16, 16)] = jnp.zeros((16,), jnp.float32)
    din_acc[pl.ds(i * 16, 16)] = jnp.zeros((16,), jnp.float32)
    return carry

  lax.fori_loop(0, N_PAD // 16, z, 0)

  base = pl.multiple_of(tile * EPT, 8)
  pltpu.sync_copy(src_hbm.at[pl.ds(base, EPT)], srcv)
  pltpu.sync_copy(dst_hbm.at[pl.ds(base, EPT)], dstv)
  ones = jnp.ones((16,), jnp.float32)

  def it(i, carry):
    plsc.addupdate_scatter(dout_acc, [srcv[pl.ds(i * 16, 16)]], ones)
    plsc.addupdate_scatter(din_acc, [dstv[pl.ds(i * 16, 16)]], ones)
    return carry

  lax.fori_loop(0, EPT // 16, it, 0)
  pltpu.sync_copy(dout_acc, dout_hbm.at[tile])
  pltpu.sync_copy(din_acc, din_hbm.at[tile])


@functools.cache
def _deg_kernel():
  return pl.kernel(
      _deg_body,
      out_type=(jax.ShapeDtypeStruct((NW, N_PAD), jnp.float32),
                jax.ShapeDtypeStruct((NW, N_PAD), jnp.float32)),
      mesh=plsc.VectorSubcoreMesh(**_MESH),
      scratch_types=[
          pltpu.VMEM((EPT,), jnp.int32),
          pltpu.VMEM((EPT,), jnp.int32),
          pltpu.VMEM((N_PAD,), jnp.float32),
          pltpu.VMEM((N_PAD,), jnp.float32),
          pltpu.SemaphoreType.DMA,
      ],
      compiler_params=pltpu.CompilerParams(needs_layout_passes=False),
  )


def _agg_body(h_hbm, src_hbm, dst_hbm, zeros_hbm, out_hbm,
              srcv, dstb, rows, acc, sems_g, sems_i):
  c = lax.axis_index("c")
  s = lax.axis_index("s")
  tile = c * NS + s
  r0 = pl.multiple_of(s * ROWS_PT, 8)
  pltpu.sync_copy(zeros_hbm.at[pl.ds(r0, ROWS_PT)], acc.at[pl.ds(r0, ROWS_PT)])
  e0 = pl.multiple_of(tile * EPT, 8)
  pltpu.sync_copy(src_hbm.at[pl.ds(e0, EPT)], srcv)
  plsc.subcore_barrier()

  def gather(i, b):
    idx = pl.multiple_of(i * CHUNK, 8)
    pltpu.async_copy(h_hbm.at[srcv.at[pl.ds(idx, CHUNK)]], rows[b], sems_g[b])

  def wait_gather(i, b):
    idx = pl.multiple_of(i * CHUNK, 8)
    pltpu.make_async_copy(h_hbm.at[srcv.at[pl.ds(idx, CHUNK)]], rows[b],
                          sems_g[b]).wait()

  def dst_fetch(i, b):
    base = pl.multiple_of(e0 + i * CHUNK, 8)
    pltpu.async_copy(dst_hbm.at[pl.ds(base, CHUNK)], dstb[b], sems_i[b])

  def wait_dst(i, b):
    base = pl.multiple_of(e0 + i * CHUNK, 8)
    pltpu.make_async_copy(dst_hbm.at[pl.ds(base, CHUNK)], dstb[b],
                          sems_i[b]).wait()

  for b in range(NBUF):
    dst_fetch(b, b)
    gather(b, b)

  def it(outer, carry):
    for b in range(NBUF):
      i = outer * NBUF + b
      wait_gather(i, b)
      wait_dst(i, b)
      pltpu.sync_copy(rows[b], acc.at[dstb[b]], add=True)
      dst_fetch(i + NBUF, b)
      gather(i + NBUF, b)
    return carry

  lax.fori_loop(0, (ITERS - NBUF) // NBUF, it, 0)
  for b in range(NBUF):
    i = ITERS - NBUF + b
    wait_gather(i, b)
    wait_dst(i, b)
    pltpu.sync_copy(rows[b], acc.at[dstb[b]], add=True)

  plsc.subcore_barrier()
  pltpu.sync_copy(acc.at[pl.ds(r0, ROWS_PT)], out_hbm.at[c, pl.ds(r0, ROWS_PT)])


@functools.cache
def _agg_kernel():
  return pl.kernel(
      _agg_body,
      out_type=jax.ShapeDtypeStruct((NC, N_PAD, D), jnp.float32),
      mesh=plsc.VectorSubcoreMesh(**_MESH),
      scratch_types=[
          pltpu.VMEM((EPT,), jnp.int32),
          [pltpu.VMEM((CHUNK,), jnp.int32)] * NBUF,
          [pltpu.VMEM((CHUNK, D), jnp.float32)] * NBUF,
          pltpu.VMEM_SHARED((N_PAD, D), jnp.float32),
          [pltpu.SemaphoreType.DMA] * NBUF,
          [pltpu.SemaphoreType.DMA] * NBUF,
      ],
  )


def _scale_from_parts(parts_ref):
  d = jnp.sum(parts_ref[...], axis=0)[:, None]
  return lax.rsqrt(jnp.maximum(d, 1.0))


def _tc_pre_body(feat_ref, dop_ref, h_ref):
  dos = _scale_from_parts(dop_ref)
  h_ref[:N] = feat_ref[...] * dos[:N]
  h_ref[N:] = jnp.zeros((N_PAD - N, D), jnp.float32)


def _tc_mid_body(p_ref, w_ref, b_ref, dop_ref, dip_ref, h_ref):
  agg = p_ref[0] + p_ref[1]
  dis = _scale_from_parts(dip_ref)
  dos = _scale_from_parts(dop_ref)
  rst = jnp.dot(agg, w_ref[...], preferred_element_type=jnp.float32)
  x = jnp.maximum(rst * dis + b_ref[...], 0.0)
  h_ref[:N] = x[:N] * dos[:N]
  h_ref[N:] = jnp.zeros((N_PAD - N, D), jnp.float32)


def _tc_post_body(p_ref, w_ref, b_ref, dip_ref,
                  wm0_ref, bm0_ref, wm1_ref, bm1_ref,
                  wm2_ref, bm2_ref, wm3_ref, bm3_ref, y_ref):
  agg = p_ref[0, :N, :] + p_ref[1, :N, :]
  dis = _scale_from_parts(dip_ref)[:N]
  rst = jnp.dot(agg, w_ref[...], preferred_element_type=jnp.float32)
  x = jnp.maximum(rst * dis + b_ref[...], 0.0)
  hg = jnp.mean(x, axis=0, keepdims=True)
  y = jnp.maximum(
      jnp.dot(hg, wm0_ref[...], preferred_element_type=jnp.float32)
      + bm0_ref[...], 0.0)
  y = jnp.maximum(
      jnp.dot(y, wm1_ref[...], preferred_element_type=jnp.float32)
      + bm1_ref[...], 0.0)
  y = jnp.maximum(
      jnp.dot(y, wm2_ref[...], preferred_element_type=jnp.float32)
      + bm2_ref[...], 0.0)
  y_ref[...] = (jnp.sum(y * wm3_ref[...].reshape(1, 16), axis=1, keepdims=True)
                + bm3_ref[...])


def kernel(feat, edge_index, W1, b1, W2, b2,
           Wm0, bm0, Wm1, bm1, Wm2, bm2, Wm3, bm3):
  pad = N + jnp.arange(E_PAD - E, dtype=jnp.int32) % (N_PAD - N)
  src = jnp.concatenate([edge_index[0], pad])
  dst = jnp.concatenate([edge_index[1], pad])
  zeros_agg = jnp.zeros((N_PAD, D), jnp.float32)

  dout_parts, din_parts = _deg_kernel()(src, dst)

  h1 = pl.pallas_call(
      _tc_pre_body,
      out_shape=jax.ShapeDtypeStruct((N_PAD, D), jnp.float32),
  )(feat, dout_parts)

  agg1 = _agg_kernel()(h1, src, dst, zeros_agg)

  h2 = pl.pallas_call(
      _tc_mid_body,
      out_shape=jax.ShapeDtypeStruct((N_PAD, D), jnp.float32),
  )(agg1, W1, b1.reshape(1, D), dout_parts, din_parts)

  agg2 = _agg_kernel()(h2, src, dst, zeros_agg)

  y = pl.pallas_call(
      _tc_post_body,
      out_shape=jax.ShapeDtypeStruct((1, 1), jnp.float32),
  )(agg2, W2, b2.reshape(1, D), din_parts,
    Wm0, bm0.reshape(1, -1), Wm1, bm1.reshape(1, -1),
    Wm2, bm2.reshape(1, -1), Wm3, bm3.reshape(1, -1))
  return y
